# baseline (speedup 1.0000x reference)
import functools

import jax
import jax.numpy as jnp
from jax import lax
from jax.experimental import pallas as pl
from jax.experimental.pallas import tpu as pltpu
from jax.experimental.pallas import tpu_sc as plsc

F32 = jnp.float32
ISQ = 1.0 / (1.0 + 1e-5) ** 0.5


_HALF = 5120
_CHUNK = 128
_RING = 4


def _sc_agg_body(edges_per_sub, hsplit, packed, zhbm, out, acc, packed_all,
                 src_bufs, dst_bufs, rows_bufs, gsems, ssems):
    c = lax.axis_index("c")
    s = lax.axis_index("s")
    rows_per_tile = _HALF // 16
    n_chunks = edges_per_sub // _CHUNK
    dummy = _HALF + (s & 7)

    pltpu.sync_copy(packed.at[pl.ds(s * edges_per_sub, edges_per_sub)],
                    packed_all)

    for p in range(2):
        base_node = p * _HALF
        pltpu.sync_copy(zhbm.at[pl.ds(0, 80)], rows_bufs[0].at[pl.ds(0, 80)])
        for z in range(rows_per_tile // 80):
            pltpu.sync_copy(
                rows_bufs[0].at[pl.ds(0, 80)],
                acc.at[pl.ds(s * rows_per_tile + z * 80, 80)])
        plsc.subcore_barrier()

        def fire(t, u):
            for k in range(_CHUNK // 16):
                w = packed_all[pl.ds(t * _CHUNK + k * 16, 16)]
                src_bufs[u][pl.ds(k * 16, 16)] = w & 0xFFFF
                d = (w >> 16) - base_node
                ok = (d >= 0) & (d < _HALF)
                dst_bufs[u][pl.ds(k * 16, 16)] = jnp.where(ok, d, dummy)
            pltpu.async_copy(hsplit.at[c].at[src_bufs[u]], rows_bufs[u],
                             gsems[u])

        def wait_gather(u):
            pltpu.make_async_copy(zhbm, rows_bufs[u], gsems[u]).wait()

        def scatter(u):
            pltpu.async_copy(rows_bufs[u], acc.at[dst_bufs[u]], ssems[u],
                             add=True)

        def wait_scatter(u):
            pltpu.make_async_copy(rows_bufs[u], acc.at[dst_bufs[u]],
                                  ssems[u]).wait()

        n_iters = (n_chunks + 2 * _RING - 1) // _RING

        def body(i, _):
            for u in range(_RING):
                t = i * _RING + u

                @pl.when((t >= _RING) & (t < n_chunks + _RING))
                def _():
                    wait_scatter(u)

                @pl.when(t < n_chunks)
                def _():
                    fire(t, u)

                us = (u - 2) % _RING

                @pl.when((t >= 2) & (t < n_chunks + 2))
                def _():
                    wait_gather(us)
                    scatter(us)

            return 0

        lax.fori_loop(0, n_iters, body, 0)
        plsc.subcore_barrier()

        for z in range(rows_per_tile // 80):
            pltpu.sync_copy(
                acc.at[pl.ds(s * rows_per_tile + z * 80, 80)],
                rows_bufs[0].at[pl.ds(0, 80)])
            pltpu.sync_copy(
                rows_bufs[0].at[pl.ds(0, 80)],
                out.at[c, pl.ds(base_node + s * rows_per_tile + z * 80, 80)])
        plsc.subcore_barrier()


def _sc_agg(hsplit, packed, zhbm):
    _, n, _ = hsplit.shape
    e = packed.shape[0]
    edges_per_sub = e // 16
    mesh = plsc.VectorSubcoreMesh(core_axis_name="c", subcore_axis_name="s")
    return pl.kernel(
        functools.partial(_sc_agg_body, edges_per_sub),
        out_type=pltpu.MemorySpace.HBM((2, 2 * _HALF, 128), F32),
        mesh=mesh,
        scratch_types=[
            pltpu.VMEM_SHARED((_HALF + 8, 128), F32),
            pltpu.VMEM((edges_per_sub,), jnp.int32),
            [pltpu.VMEM((_CHUNK,), jnp.int32) for _ in range(_RING)],
            [pltpu.VMEM((_CHUNK,), jnp.int32) for _ in range(_RING)],
            [pltpu.VMEM((_CHUNK, 128), F32) for _ in range(_RING)],
            [pltpu.SemaphoreType.DMA for _ in range(_RING)],
            [pltpu.SemaphoreType.DMA for _ in range(_RING)],
        ],
    )(hsplit, packed, zhbm)


def _in_mlp_body(x_ref, w_ref, b_ref, h_ref, hs_ref):
    h = jnp.maximum(
        jnp.dot(x_ref[...], w_ref[...], preferred_element_type=F32) + b_ref[...], 0.0)
    h_ref[...] = h
    hs_ref[0] = h[:, :128]
    hs_ref[1] = h[:, 128:]


def _gin_mlp_body(h_ref, a_ref, eps_ref, w1_ref, b1_ref, g1_ref, be1_ref,
                  w2_ref, b2_ref, g2_ref, be2_ref, ho_ref, hs_ref):
    agg = jnp.concatenate([a_ref[0], a_ref[1]], axis=1)
    z = (1.0 + eps_ref[0]) * h_ref[...] + agg
    z = jnp.dot(z, w1_ref[...], preferred_element_type=F32) + b1_ref[...]
    z = jnp.maximum(z * (ISQ * g1_ref[...]) + be1_ref[...], 0.0)
    z = jnp.dot(z, w2_ref[...], preferred_element_type=F32) + b2_ref[...]
    h = jnp.maximum(z * (ISQ * g2_ref[...]) + be2_ref[...], 0.0)
    ho_ref[...] = h
    hs_ref[0] = h[:, :128]
    hs_ref[1] = h[:, 128:]


def _l2norm(t):
    n2 = jnp.sum(t * t, axis=1, keepdims=True)
    return t / jnp.maximum(jnp.sqrt(n2), 1e-12)


def _head_body(h_ref, bb_ref, wf_ref, bf_ref, ef_ref, hg_ref, cnt_ref):
    i = pl.program_id(0)
    h = h_ref[...]
    ef_ref[...] = _l2norm(
        jnp.dot(h, wf_ref[...], preferred_element_type=F32) + bf_ref[...])
    rows = h.shape[0]
    ids = jax.lax.broadcasted_iota(jnp.int32, (128, rows), 0)
    oh = jnp.where(ids == bb_ref[0], 1.0, 0.0).astype(F32)
    pg = jnp.dot(oh, h, preferred_element_type=F32)
    cg = jnp.sum(oh, axis=1, keepdims=True)

    @pl.when(i == 0)
    def _():
        hg_ref[...] = pg
        cnt_ref[...] = cg

    @pl.when(i > 0)
    def _():
        hg_ref[...] += pg
        cnt_ref[...] += cg


def _mole_body(hg_ref, cnt_ref, wm_ref, bm_ref, out_ref):
    hg = hg_ref[...] / jnp.maximum(cnt_ref[...], 1.0)
    out_ref[...] = _l2norm(
        jnp.dot(hg, wm_ref[...], preferred_element_type=F32) + bm_ref[...])


_BLK = 1000


def _in_mlp(x, w, b):
    n, d_in = x.shape
    d_h = w.shape[1]
    grid = n // _BLK
    return pl.pallas_call(
        _in_mlp_body,
        grid=(grid,),
        in_specs=[
            pl.BlockSpec((_BLK, d_in), lambda i: (i, 0)),
            pl.BlockSpec((d_in, d_h), lambda i: (0, 0)),
            pl.BlockSpec((1, d_h), lambda i: (0, 0)),
        ],
        out_specs=[
            pl.BlockSpec((_BLK, d_h), lambda i: (i, 0)),
            pl.BlockSpec((2, _BLK, 128), lambda i: (0, i, 0)),
        ],
        out_shape=[
            jax.ShapeDtypeStruct((n, d_h), F32),
            jax.ShapeDtypeStruct((2, n, 128), F32),
        ],
    )(x, w, b)


def _gin_mlp(h, agg2, eps, w1, b1, g1, be1, w2, b2, g2, be2):
    n, d_h = h.shape
    grid = n // _BLK
    wspec = pl.BlockSpec((d_h, d_h), lambda i: (0, 0))
    vspec = pl.BlockSpec((1, d_h), lambda i: (0, 0))
    return pl.pallas_call(
        _gin_mlp_body,
        grid=(grid,),
        in_specs=[
            pl.BlockSpec((_BLK, d_h), lambda i: (i, 0)),
            pl.BlockSpec((2, _BLK, 128), lambda i: (0, i, 0)),
            pl.BlockSpec(memory_space=pltpu.MemorySpace.SMEM),
            wspec, vspec, vspec, vspec, wspec, vspec, vspec, vspec,
        ],
        out_specs=[
            pl.BlockSpec((_BLK, d_h), lambda i: (i, 0)),
            pl.BlockSpec((2, _BLK, 128), lambda i: (0, i, 0)),
        ],
        out_shape=[
            jax.ShapeDtypeStruct((n, d_h), F32),
            jax.ShapeDtypeStruct((2, n, 128), F32),
        ],
    )(h, agg2, eps, w1, b1, g1, be1, w2, b2, g2, be2)


def _head(h, batch3, wf, bf):
    n, d_h = h.shape
    d_e = wf.shape[1]
    grid = n // _BLK
    return pl.pallas_call(
        _head_body,
        grid=(grid,),
        in_specs=[
            pl.BlockSpec((_BLK, d_h), lambda i: (i, 0)),
            pl.BlockSpec((1, 1, _BLK), lambda i: (i, 0, 0)),
            pl.BlockSpec((d_h, d_e), lambda i: (0, 0)),
            pl.BlockSpec((1, d_e), lambda i: (0, 0)),
        ],
        out_specs=[
            pl.BlockSpec((_BLK, d_e), lambda i: (i, 0)),
            pl.BlockSpec((128, d_h), lambda i: (0, 0)),
            pl.BlockSpec((128, 1), lambda i: (0, 0)),
        ],
        out_shape=[
            jax.ShapeDtypeStruct((n, d_e), F32),
            jax.ShapeDtypeStruct((128, d_h), F32),
            jax.ShapeDtypeStruct((128, 1), F32),
        ],
    )(h, batch3, wf, bf)


def _mole(hg, cnt, wm, bm):
    d_h = hg.shape[1]
    d_e = wm.shape[1]
    return pl.pallas_call(
        _mole_body,
        out_shape=jax.ShapeDtypeStruct((128, d_e), F32),
    )(hg, cnt, wm, bm)


def kernel(x, edge_index, batch, W_in, b_in, W1, b1, g1, be1, W2, b2, eps_gin,
           g2, be2, W_frag, b_frag, W_mole, b_mole):
    n = x.shape[0]
    packed = jnp.bitwise_or(edge_index[0],
                            jnp.left_shift(edge_index[1], 16))
    eps = packed.shape[0] // 16
    pad = (-eps) % _CHUNK
    if pad:
        spill_blk = jnp.broadcast_to(
            jnp.arange(pad, dtype=jnp.int32) | (20000 << 16), (16, pad))
        packed = jnp.concatenate(
            [packed.reshape(16, eps), spill_blk], axis=1).reshape(-1)
    batch3 = batch.reshape(n // _BLK, 1, _BLK)

    h, hsplit = _in_mlp(x, W_in, b_in.reshape(1, -1))
    hs = [h]
    zhbm = jnp.zeros((_CHUNK, 128), F32)
    for i in range(3):
        agg2 = _sc_agg(hsplit, packed, zhbm)
        h, hsplit = _gin_mlp(h, agg2, eps_gin[i].reshape(1),
                             W1[i], b1[i].reshape(1, -1), g1[i].reshape(1, -1),
                             be1[i].reshape(1, -1), W2[i], b2[i].reshape(1, -1),
                             g2[i].reshape(1, -1), be2[i].reshape(1, -1))
        hs.append(h)

    emb_frag, hg, cnt = _head(h, batch3, W_frag, b_frag.reshape(1, -1))
    emb_mole = _mole(hg, cnt, W_mole, b_mole.reshape(1, -1))
    return (emb_mole, emb_frag, jnp.stack(hs))


# speedup vs baseline: 1.0360x; 1.0360x over previous
import functools

import jax
import jax.numpy as jnp
from jax import lax
from jax.experimental import pallas as pl
from jax.experimental.pallas import tpu as pltpu
from jax.experimental.pallas import tpu_sc as plsc

F32 = jnp.float32
ISQ = 1.0 / (1.0 + 1e-5) ** 0.5


_HALF = 5120
_CHUNK = 80
_RING = 5


def _sc_agg_body(edges_per_sub, hsplit, packed, zhbm, out, acc, packed_all,
                 src_bufs, dst_bufs, rows_bufs, gsems, ssems):
    c = lax.axis_index("c")
    s = lax.axis_index("s")
    rows_per_tile = _HALF // 16
    n_chunks = edges_per_sub // _CHUNK
    dummy = _HALF + (s & 7)

    pltpu.sync_copy(packed.at[pl.ds(s * edges_per_sub, edges_per_sub)],
                    packed_all)

    for p in range(2):
        base_node = p * _HALF
        pltpu.sync_copy(zhbm, rows_bufs[0])
        for z in range(rows_per_tile // _CHUNK):
            pltpu.sync_copy(
                rows_bufs[0],
                acc.at[pl.ds(s * rows_per_tile + z * _CHUNK, _CHUNK)])
        plsc.subcore_barrier()

        def fire(t, u):
            for k in range(_CHUNK // 16):
                w = packed_all[pl.ds(t * _CHUNK + k * 16, 16)]
                src_bufs[u][pl.ds(k * 16, 16)] = w & 0xFFFF
                d = (w >> 16) - base_node
                ok = (d >= 0) & (d < _HALF)
                dst_bufs[u][pl.ds(k * 16, 16)] = jnp.where(ok, d, dummy)
            pltpu.async_copy(hsplit.at[c].at[src_bufs[u]], rows_bufs[u],
                             gsems[u])

        def wait_gather(u):
            pltpu.make_async_copy(zhbm, rows_bufs[u], gsems[u]).wait()

        def scatter(u):
            pltpu.async_copy(rows_bufs[u], acc.at[dst_bufs[u]], ssems[u],
                             add=True)

        def wait_scatter(u):
            pltpu.make_async_copy(rows_bufs[u], acc.at[dst_bufs[u]],
                                  ssems[u]).wait()

        n_iters = n_chunks // _RING + 1

        def body(i, _):
            for u in range(_RING):
                t = i * _RING + u

                @pl.when((t >= _RING) & (t < n_chunks + _RING))
                def _():
                    wait_scatter(u)

                @pl.when(t < n_chunks)
                def _():
                    fire(t, u)

                us = (u - 3) % _RING

                @pl.when((t >= 3) & (t < n_chunks + 3))
                def _():
                    wait_gather(us)
                    scatter(us)

            return 0

        lax.fori_loop(0, n_iters, body, 0)
        plsc.subcore_barrier()

        for z in range(rows_per_tile // _CHUNK):
            pltpu.sync_copy(
                acc.at[pl.ds(s * rows_per_tile + z * _CHUNK, _CHUNK)],
                rows_bufs[0])
            pltpu.sync_copy(
                rows_bufs[0],
                out.at[c, pl.ds(base_node + s * rows_per_tile + z * _CHUNK,
                                _CHUNK)])
        plsc.subcore_barrier()


def _sc_agg(hsplit, packed, zhbm):
    _, n, _ = hsplit.shape
    e = packed.shape[0]
    edges_per_sub = e // 16
    mesh = plsc.VectorSubcoreMesh(core_axis_name="c", subcore_axis_name="s")
    return pl.kernel(
        functools.partial(_sc_agg_body, edges_per_sub),
        out_type=pltpu.MemorySpace.HBM((2, 2 * _HALF, 128), F32),
        mesh=mesh,
        scratch_types=[
            pltpu.VMEM_SHARED((_HALF + 8, 128), F32),
            pltpu.VMEM((edges_per_sub,), jnp.int32),
            [pltpu.VMEM((_CHUNK,), jnp.int32) for _ in range(_RING)],
            [pltpu.VMEM((_CHUNK,), jnp.int32) for _ in range(_RING)],
            [pltpu.VMEM((_CHUNK, 128), F32) for _ in range(_RING)],
            [pltpu.SemaphoreType.DMA for _ in range(_RING)],
            [pltpu.SemaphoreType.DMA for _ in range(_RING)],
        ],
    )(hsplit, packed, zhbm)


def _in_mlp_body(x_ref, w_ref, b_ref, h_ref, hs_ref):
    h = jnp.maximum(
        jnp.dot(x_ref[...], w_ref[...], preferred_element_type=F32) + b_ref[...], 0.0)
    h_ref[...] = h
    hs_ref[0] = h[:, :128]
    hs_ref[1] = h[:, 128:]


def _gin_mlp_body(h_ref, a_ref, eps_ref, w1_ref, b1_ref, g1_ref, be1_ref,
                  w2_ref, b2_ref, g2_ref, be2_ref, ho_ref, hs_ref):
    agg = jnp.concatenate([a_ref[0], a_ref[1]], axis=1)
    z = (1.0 + eps_ref[0]) * h_ref[...] + agg
    z = jnp.dot(z, w1_ref[...], preferred_element_type=F32) + b1_ref[...]
    z = jnp.maximum(z * (ISQ * g1_ref[...]) + be1_ref[...], 0.0)
    z = jnp.dot(z, w2_ref[...], preferred_element_type=F32) + b2_ref[...]
    h = jnp.maximum(z * (ISQ * g2_ref[...]) + be2_ref[...], 0.0)
    ho_ref[...] = h
    hs_ref[0] = h[:, :128]
    hs_ref[1] = h[:, 128:]


def _l2norm(t):
    n2 = jnp.sum(t * t, axis=1, keepdims=True)
    return t / jnp.maximum(jnp.sqrt(n2), 1e-12)


def _head_body(h_ref, bb_ref, wf_ref, bf_ref, ef_ref, hg_ref, cnt_ref):
    i = pl.program_id(0)
    h = h_ref[...]
    ef_ref[...] = _l2norm(
        jnp.dot(h, wf_ref[...], preferred_element_type=F32) + bf_ref[...])
    rows = h.shape[0]
    ids = jax.lax.broadcasted_iota(jnp.int32, (128, rows), 0)
    oh = jnp.where(ids == bb_ref[0], 1.0, 0.0).astype(F32)
    pg = jnp.dot(oh, h, preferred_element_type=F32)
    cg = jnp.sum(oh, axis=1, keepdims=True)

    @pl.when(i == 0)
    def _():
        hg_ref[...] = pg
        cnt_ref[...] = cg

    @pl.when(i > 0)
    def _():
        hg_ref[...] += pg
        cnt_ref[...] += cg


def _mole_body(hg_ref, cnt_ref, wm_ref, bm_ref, out_ref):
    hg = hg_ref[...] / jnp.maximum(cnt_ref[...], 1.0)
    out_ref[...] = _l2norm(
        jnp.dot(hg, wm_ref[...], preferred_element_type=F32) + bm_ref[...])


_BLK = 1000


def _in_mlp(x, w, b):
    n, d_in = x.shape
    d_h = w.shape[1]
    grid = n // _BLK
    return pl.pallas_call(
        _in_mlp_body,
        grid=(grid,),
        in_specs=[
            pl.BlockSpec((_BLK, d_in), lambda i: (i, 0)),
            pl.BlockSpec((d_in, d_h), lambda i: (0, 0)),
            pl.BlockSpec((1, d_h), lambda i: (0, 0)),
        ],
        out_specs=[
            pl.BlockSpec((_BLK, d_h), lambda i: (i, 0)),
            pl.BlockSpec((2, _BLK, 128), lambda i: (0, i, 0)),
        ],
        out_shape=[
            jax.ShapeDtypeStruct((n, d_h), F32),
            jax.ShapeDtypeStruct((2, n, 128), F32),
        ],
    )(x, w, b)


def _gin_mlp(h, agg2, eps, w1, b1, g1, be1, w2, b2, g2, be2):
    n, d_h = h.shape
    grid = n // _BLK
    wspec = pl.BlockSpec((d_h, d_h), lambda i: (0, 0))
    vspec = pl.BlockSpec((1, d_h), lambda i: (0, 0))
    return pl.pallas_call(
        _gin_mlp_body,
        grid=(grid,),
        in_specs=[
            pl.BlockSpec((_BLK, d_h), lambda i: (i, 0)),
            pl.BlockSpec((2, _BLK, 128), lambda i: (0, i, 0)),
            pl.BlockSpec(memory_space=pltpu.MemorySpace.SMEM),
            wspec, vspec, vspec, vspec, wspec, vspec, vspec, vspec,
        ],
        out_specs=[
            pl.BlockSpec((_BLK, d_h), lambda i: (i, 0)),
            pl.BlockSpec((2, _BLK, 128), lambda i: (0, i, 0)),
        ],
        out_shape=[
            jax.ShapeDtypeStruct((n, d_h), F32),
            jax.ShapeDtypeStruct((2, n, 128), F32),
        ],
    )(h, agg2, eps, w1, b1, g1, be1, w2, b2, g2, be2)


def _head(h, batch3, wf, bf):
    n, d_h = h.shape
    d_e = wf.shape[1]
    grid = n // _BLK
    return pl.pallas_call(
        _head_body,
        grid=(grid,),
        in_specs=[
            pl.BlockSpec((_BLK, d_h), lambda i: (i, 0)),
            pl.BlockSpec((1, 1, _BLK), lambda i: (i, 0, 0)),
            pl.BlockSpec((d_h, d_e), lambda i: (0, 0)),
            pl.BlockSpec((1, d_e), lambda i: (0, 0)),
        ],
        out_specs=[
            pl.BlockSpec((_BLK, d_e), lambda i: (i, 0)),
            pl.BlockSpec((128, d_h), lambda i: (0, 0)),
            pl.BlockSpec((128, 1), lambda i: (0, 0)),
        ],
        out_shape=[
            jax.ShapeDtypeStruct((n, d_e), F32),
            jax.ShapeDtypeStruct((128, d_h), F32),
            jax.ShapeDtypeStruct((128, 1), F32),
        ],
    )(h, batch3, wf, bf)


def _mole(hg, cnt, wm, bm):
    d_h = hg.shape[1]
    d_e = wm.shape[1]
    return pl.pallas_call(
        _mole_body,
        out_shape=jax.ShapeDtypeStruct((128, d_e), F32),
    )(hg, cnt, wm, bm)


def kernel(x, edge_index, batch, W_in, b_in, W1, b1, g1, be1, W2, b2, eps_gin,
           g2, be2, W_frag, b_frag, W_mole, b_mole):
    n = x.shape[0]
    packed = jnp.bitwise_or(edge_index[0],
                            jnp.left_shift(edge_index[1], 16))
    batch3 = batch.reshape(n // _BLK, 1, _BLK)

    h, hsplit = _in_mlp(x, W_in, b_in.reshape(1, -1))
    hs = [h]
    zhbm = jnp.zeros((_CHUNK, 128), F32)
    for i in range(3):
        agg2 = _sc_agg(hsplit, packed, zhbm)
        h, hsplit = _gin_mlp(h, agg2, eps_gin[i].reshape(1),
                             W1[i], b1[i].reshape(1, -1), g1[i].reshape(1, -1),
                             be1[i].reshape(1, -1), W2[i], b2[i].reshape(1, -1),
                             g2[i].reshape(1, -1), be2[i].reshape(1, -1))
        hs.append(h)

    emb_frag, hg, cnt = _head(h, batch3, W_frag, b_frag.reshape(1, -1))
    emb_mole = _mole(hg, cnt, W_mole, b_mole.reshape(1, -1))
    return (emb_mole, emb_frag, jnp.stack(hs))
